# baseline (speedup 1.0000x reference)
import functools
import math

import jax
import jax.numpy as jnp
from jax import lax
from jax.experimental import pallas as pl
from jax.experimental.pallas import tpu as pltpu, tpu_sc as plsc

V = 100000
SMOOTH = 0.1
CONF = 1.0 - SMOOTH
EPS = SMOOTH / (V - 2)
K_ROW = CONF * math.log(CONF) + (V - 2) * EPS * math.log(EPS)

N_ROWS = 1024
R0 = 512
NR = N_ROWS - R0
CB = 2560
C_SC = 98304
TAILW = 2048


def _weighted_sum(x, c, col0, width_limit):
    col = jax.lax.broadcasted_iota(jnp.int32, x.shape, 1) + col0
    coeff = jnp.where(col.astype(jnp.float32) == c, -CONF, -EPS)
    xz = jnp.where(col < width_limit, x, 0.0)
    return jnp.sum(coeff * xz, axis=1, keepdims=True)


def _tc_body(toka_ref, tokb_ref, xa_ref, xb_ref, tail_ref, a_ref, tt_ref):
    j = pl.program_id(0)
    c = toka_ref[...]
    notpad = (c != 0.0).astype(jnp.float32)
    term = (_weighted_sum(xa_ref[...], c, 2 * j * CB, V)
            + _weighted_sum(xb_ref[...], c, (2 * j + 1) * CB, V))
    contrib = jnp.sum(notpad * term)
    extra = jnp.sum(notpad * (K_ROW + EPS * xa_ref[:, 0:1]))
    contrib = contrib + jnp.where(j == 0, extra, 0.0)

    @pl.when(j == 0)
    def _init():
        a_ref[...] = jnp.zeros((1, 1), jnp.float32)
        cB = tokb_ref[...]
        tt_ref[...] = _weighted_sum(tail_ref[...], cB, 48 * TAILW, V)

    a_ref[...] += jnp.full((1, 1), contrib, jnp.float32)


def _tc_pass(tok_col, x):
    grid = (pl.cdiv(V, CB) // 2,)
    a, tt = pl.pallas_call(
        _tc_body,
        grid=grid,
        in_specs=[
            pl.BlockSpec((R0, 1), lambda j: (0, 0)),
            pl.BlockSpec((NR, 1), lambda j: (R0 // NR, 0)),
            pl.BlockSpec((R0, CB), lambda j: (0, 2 * j)),
            pl.BlockSpec((R0, CB), lambda j: (0, 2 * j + 1)),
            pl.BlockSpec((NR, TAILW), lambda j: (R0 // NR, 48)),
        ],
        out_specs=[
            pl.BlockSpec((1, 1), lambda j: (0, 0)),
            pl.BlockSpec((NR, 1), lambda j: (0, 0)),
        ],
        out_shape=[
            jax.ShapeDtypeStruct((1, 1), jnp.float32),
            jax.ShapeDtypeStruct((NR, 1), jnp.float32),
        ],
    )(tok_col, tok_col, x, x, x)
    return a[0, 0], tt[:, 0]


_SC_INFO = plsc.get_sparse_core_info()
_NC, _NS = _SC_INFO.num_cores, _SC_INFO.num_subcores
_NW = _NC * _NS
_RPW = NR // _NW
_BANDS = _RPW // 8
CW = 1024
_NCH = C_SC // CW

_sc_mesh = plsc.VectorSubcoreMesh(core_axis_name="c", subcore_axis_name="s")


@functools.partial(
    pl.kernel,
    mesh=_sc_mesh,
    out_type=[
        jax.ShapeDtypeStruct((_NW, 16), jnp.float32),
        jax.ShapeDtypeStruct((NR, 16), jnp.float32),
        jax.ShapeDtypeStruct((NR, 16), jnp.float32),
    ],
    scratch_types=[
        pltpu.VMEM((_RPW,), jnp.int32),
        pltpu.VMEM((8, CW), jnp.float32),
        pltpu.VMEM((8, CW), jnp.float32),
        pltpu.VMEM((16,), jnp.float32),
        pltpu.VMEM((_RPW, 16), jnp.float32),
        pltpu.VMEM((_RPW, 16), jnp.float32),
        pltpu.SemaphoreType.DMA,
        pltpu.SemaphoreType.DMA,
    ],
)
def _sc_pass(x_hbm, tok_hbm, s_hbm, g_hbm, x0_hbm,
             tok_v, buf0, buf1, acc_v, g_v, x0_v, sem0, sem1):
    wid = lax.axis_index("s") * _NC + lax.axis_index("c")
    row0 = R0 + wid * _RPW
    zeros16 = jnp.zeros((16,), jnp.float32)
    lane = lax.iota(jnp.int32, 16)

    pltpu.sync_copy(tok_hbm.at[pl.ds(row0, _RPW)], tok_v)
    tok16 = tok_v[...]
    acc_v[...] = zeros16
    for k in range(_RPW):
        g_v.at[k][...] = zeros16
        x0_v.at[k][...] = zeros16

    def band(b):
        r0 = row0 + 8 * b
        bufs = (buf0, buf1)
        sems = (sem0, sem1)

        def chunk_copy(c, buf, sem):
            return pltpu.make_async_copy(
                x_hbm.at[pl.ds(r0, 8), pl.ds(c * CW, CW)], buf, sem)

        chunk_copy(0, buf0, sem0).start()
        chunk_copy(1, buf1, sem1).start()

        def process(buf, c):
            for r in range(8):
                k = 8 * b + r
                ck = tok16[k]

                @pl.when(ck != 0)
                def _():
                    a0 = buf[r, pl.ds(0, 16)]
                    a1 = buf[r, pl.ds(16, 16)]
                    a2 = buf[r, pl.ds(32, 16)]
                    a3 = buf[r, pl.ds(48, 16)]
                    for t in range(1, CW // 64):
                        a0 += buf[r, pl.ds(64 * t, 16)]
                        a1 += buf[r, pl.ds(64 * t + 16, 16)]
                        a2 += buf[r, pl.ds(64 * t + 32, 16)]
                        a3 += buf[r, pl.ds(64 * t + 48, 16)]
                    acc_v[...] += (a0 + a1) + (a2 + a3)

                @pl.when((ck >= c * CW) & (ck < c * CW + CW))
                def _():
                    off = ck - c * CW
                    vec = buf[r, pl.ds((off // 16) * 16, 16)]
                    g_v.at[k][...] = jnp.where(lane == off % 16, vec, 0.0)

                @pl.when(c == 0)
                def _():
                    vec0 = buf[r, pl.ds(0, 16)]
                    x0_v.at[k][...] = jnp.where(lane == 0, vec0, 0.0)

        def loop_body(m, _):
            c = 2 * m
            pltpu.make_async_copy(
                x_hbm.at[pl.ds(r0, 8), pl.ds(0, CW)], buf0, sem0).wait()
            process(buf0, c)

            @pl.when(c + 2 < _NCH)
            def _():
                chunk_copy(c + 2, buf0, sem0).start()

            pltpu.make_async_copy(
                x_hbm.at[pl.ds(r0, 8), pl.ds(0, CW)], buf1, sem1).wait()
            process(buf1, c + 1)

            @pl.when(c + 3 < _NCH)
            def _():
                chunk_copy(c + 3, buf1, sem1).start()

            return 0

        lax.fori_loop(0, _NCH // 2, loop_body, 0)

    for b in range(_BANDS):
        band(b)

    pltpu.sync_copy(acc_v, s_hbm.at[wid])
    pltpu.sync_copy(g_v, g_hbm.at[pl.ds(wid * _RPW, _RPW)])
    pltpu.sync_copy(x0_v, x0_hbm.at[pl.ds(wid * _RPW, _RPW)])


def kernel(predicted_log_probabilities, tgt_tokens):
    n, v = predicted_log_probabilities.shape
    x = predicted_log_probabilities
    tok_col = tgt_tokens.reshape(n, 1).astype(jnp.float32)
    s16, g16, x016 = _sc_pass(x, tgt_tokens)
    a, tt = _tc_pass(tok_col, x)
    notpad = (tgt_tokens[R0:] != 0).astype(jnp.float32)
    g = jnp.sum(g16, axis=1)
    x0 = jnp.sum(x016, axis=1)
    loss_sc = (K_ROW * jnp.sum(notpad)
               - EPS * jnp.sum(s16)
               + jnp.sum(notpad * (EPS * x0 - (CONF - EPS) * g + tt)))
    return a + loss_sc


# speedup vs baseline: 1.3114x; 1.3114x over previous
import math

import jax
import jax.numpy as jnp
from jax.experimental import pallas as pl

V = 100000
SMOOTH = 0.1
CONF = 1.0 - SMOOTH
EPS = SMOOTH / (V - 2)
K_ROW = CONF * math.log(CONF) + (V - 2) * EPS * math.log(EPS)

RB = 1024
CB = 2560


def _weighted_sum(x, c, j_block):
    col = jax.lax.broadcasted_iota(jnp.int32, x.shape, 1) + j_block * CB
    coeff = jnp.where(col.astype(jnp.float32) == c, -CONF, -EPS)
    xz = jnp.where(col < V, x, 0.0)
    return jnp.sum(coeff * xz, axis=1, keepdims=True)


def _loss_body(tok_ref, xa_ref, xb_ref, out_ref):
    j = pl.program_id(0)
    c = tok_ref[...]
    notpad = (c != 0.0).astype(jnp.float32)
    term = _weighted_sum(xa_ref[...], c, 2 * j) + _weighted_sum(xb_ref[...], c, 2 * j + 1)
    contrib = jnp.sum(notpad * term)
    extra = jnp.sum(notpad * (K_ROW + EPS * xa_ref[:, 0:1]))
    contrib = contrib + jnp.where(j == 0, extra, 0.0)

    @pl.when(j == 0)
    def _init():
        out_ref[...] = jnp.zeros((1, 1), jnp.float32)

    out_ref[...] += jnp.full((1, 1), contrib, jnp.float32)


def kernel(predicted_log_probabilities, tgt_tokens):
    n, v = predicted_log_probabilities.shape
    tok_col = tgt_tokens.reshape(n, 1).astype(jnp.float32)
    grid = (pl.cdiv(v, CB) // 2,)
    out = pl.pallas_call(
        _loss_body,
        grid=grid,
        in_specs=[
            pl.BlockSpec((RB, 1), lambda j: (0, 0)),
            pl.BlockSpec((RB, CB), lambda j: (0, 2 * j)),
            pl.BlockSpec((RB, CB), lambda j: (0, 2 * j + 1)),
        ],
        out_specs=pl.BlockSpec((1, 1), lambda j: (0, 0)),
        out_shape=jax.ShapeDtypeStruct((1, 1), jnp.float32),
    )(tok_col, predicted_log_probabilities, predicted_log_probabilities)
    return out[0, 0]


# speedup vs baseline: 3.8433x; 2.9306x over previous
import functools
import math

import jax
import jax.numpy as jnp
from jax import lax
from jax.experimental import pallas as pl
from jax.experimental.pallas import tpu as pltpu, tpu_sc as plsc

V = 100000
N_ROWS = 1024
SMOOTH = 0.1
CONF = 1.0 - SMOOTH
EPS = SMOOTH / (V - 2)
K_ROW = CONF * math.log(CONF) + (V - 2) * EPS * math.log(EPS)

V_TC = 79520
VB = 2840


def _tc_body(tok_ref, xt_ref, out_ref):
    j = pl.program_id(0)
    c = tok_ref[...]
    notpad = (c != 0.0).astype(jnp.float32)
    xt = xt_ref[...]
    contrib = jnp.sum((-EPS * notpad) * xt)
    extra = jnp.sum(notpad * (K_ROW + EPS * xt[0:1, :]))
    contrib = contrib + jnp.where(j == 0, extra, 0.0)

    @pl.when(j == 0)
    def _init():
        out_ref[...] = jnp.zeros((1, 1), jnp.float32)

    out_ref[...] += jnp.full((1, 1), contrib, jnp.float32)


def _tc_pass(tok_row, xt):
    out = pl.pallas_call(
        _tc_body,
        grid=(V_TC // VB,),
        in_specs=[
            pl.BlockSpec((1, N_ROWS), lambda j: (0, 0)),
            pl.BlockSpec((VB, N_ROWS), lambda j: (j, 0)),
        ],
        out_specs=pl.BlockSpec((1, 1), lambda j: (0, 0)),
        out_shape=jax.ShapeDtypeStruct((1, 1), jnp.float32),
    )(tok_row, xt)
    return out[0, 0]


_SC_INFO = plsc.get_sparse_core_info()
_NC, _NS = _SC_INFO.num_cores, _SC_INFO.num_subcores
_NW = _NC * _NS
_BPW = N_ROWS // _NW
_V_SC = V - V_TC
_RPW = _V_SC // _NW
_NBANDS = _RPW // 8

_sc_mesh = plsc.VectorSubcoreMesh(core_axis_name="c", subcore_axis_name="s")


@functools.partial(
    pl.kernel,
    mesh=_sc_mesh,
    out_type=[
        jax.ShapeDtypeStruct((N_ROWS, 16), jnp.float32),
        jax.ShapeDtypeStruct((_NW, N_ROWS), jnp.float32),
    ],
    scratch_types=[
        pltpu.VMEM((_BPW,), jnp.int32),
        pltpu.VMEM((_BPW, N_ROWS), jnp.float32),
        pltpu.VMEM((_BPW, 16), jnp.float32),
        pltpu.VMEM((8, N_ROWS), jnp.float32),
        pltpu.VMEM((8, N_ROWS), jnp.float32),
        pltpu.VMEM((N_ROWS,), jnp.float32),
        pltpu.SemaphoreType.DMA,
        pltpu.SemaphoreType.DMA,
        pltpu.SemaphoreType.DMA,
    ],
)
def _sc_pass(xt_hbm, tok_hbm, g_hbm, cs_hbm,
             tok_v, rows_v, g_v, buf0, buf1, acc_v, gsem, sem0, sem1):
    wid = lax.axis_index("s") * _NC + lax.axis_index("c")
    base = wid * _BPW
    lane = lax.iota(jnp.int32, 16)

    pltpu.sync_copy(tok_hbm.at[pl.ds(base, _BPW)], tok_v)
    gather = pltpu.async_copy(xt_hbm.at[tok_v], rows_v, gsem)

    row0 = V_TC + wid * _RPW
    for cc in range(N_ROWS // 16):
        acc_v[pl.ds(16 * cc, 16)] = jnp.zeros((16,), jnp.float32)

    def chunk_copy(b, buf, sem):
        return pltpu.make_async_copy(
            xt_hbm.at[pl.ds(row0 + 8 * b, 8), :], buf, sem)

    chunk_copy(0, buf0, sem0).start()
    chunk_copy(1, buf1, sem1).start()

    def accumulate(buf):
        for cc in range(N_ROWS // 16):
            s = buf[0, pl.ds(16 * cc, 16)]
            for r in range(1, 8):
                s += buf[r, pl.ds(16 * cc, 16)]
            acc_v[pl.ds(16 * cc, 16)] += s

    def loop_body(m, _):
        b = 2 * m
        pltpu.make_async_copy(
            xt_hbm.at[pl.ds(row0, 8), :], buf0, sem0).wait()
        accumulate(buf0)

        @pl.when(b + 2 < _NBANDS)
        def _():
            chunk_copy(b + 2, buf0, sem0).start()

        pltpu.make_async_copy(
            xt_hbm.at[pl.ds(row0, 8), :], buf1, sem1).wait()
        accumulate(buf1)

        @pl.when(b + 3 < _NBANDS)
        def _():
            chunk_copy(b + 3, buf1, sem1).start()

        return 0

    lax.fori_loop(0, _NBANDS // 2, loop_body, 0)
    pltpu.sync_copy(acc_v, cs_hbm.at[wid])

    gather.wait()
    for k in range(_BPW):
        i_col = base + k
        vec = rows_v[k, pl.ds((i_col // 16) * 16, 16)]
        g_v.at[k][...] = jnp.where(lane == i_col % 16, vec, 0.0)
    pltpu.sync_copy(g_v, g_hbm.at[pl.ds(base, _BPW)])


def kernel(predicted_log_probabilities, tgt_tokens):
    n, v = predicted_log_probabilities.shape
    xt = predicted_log_probabilities.T
    tok_row = tgt_tokens.reshape(1, n).astype(jnp.float32)
    g16, cs = _sc_pass(xt, tgt_tokens)
    a = _tc_pass(tok_row, xt)
    notpad = (tgt_tokens != 0).astype(jnp.float32)
    g = jnp.sum(g16, axis=1)
    colsum = jnp.sum(cs, axis=0)
    return (a
            - EPS * jnp.sum(notpad * colsum)
            - (CONF - EPS) * jnp.sum(notpad * g))


# speedup vs baseline: 3.8753x; 1.0083x over previous
import functools
import math

import jax
import jax.numpy as jnp
from jax import lax
from jax.experimental import pallas as pl
from jax.experimental.pallas import tpu as pltpu, tpu_sc as plsc

V = 100000
N_ROWS = 1024
SMOOTH = 0.1
CONF = 1.0 - SMOOTH
EPS = SMOOTH / (V - 2)
K_ROW = CONF * math.log(CONF) + (V - 2) * EPS * math.log(EPS)

V_TC = 79520
VB = 2840


def _tc_body(tok_ref, xt_ref, out_ref):
    j = pl.program_id(0)
    c = tok_ref[...]
    notpad = (c != 0.0).astype(jnp.float32)
    xt = xt_ref[...]
    contrib = jnp.sum((-EPS * notpad) * xt)
    extra = jnp.sum(notpad * (K_ROW + EPS * xt[0:1, :]))
    contrib = contrib + jnp.where(j == 0, extra, 0.0)

    @pl.when(j == 0)
    def _init():
        out_ref[...] = jnp.zeros((1, 1), jnp.float32)

    out_ref[...] += jnp.full((1, 1), contrib, jnp.float32)


def _tc_pass(tok_row, xt):
    out = pl.pallas_call(
        _tc_body,
        grid=(V_TC // VB,),
        in_specs=[
            pl.BlockSpec((1, N_ROWS), lambda j: (0, 0)),
            pl.BlockSpec((VB, N_ROWS), lambda j: (j, 0)),
        ],
        out_specs=pl.BlockSpec((1, 1), lambda j: (0, 0)),
        out_shape=jax.ShapeDtypeStruct((1, 1), jnp.float32),
    )(tok_row, xt)
    return out[0, 0]


_SC_INFO = plsc.get_sparse_core_info()
_NC, _NS = _SC_INFO.num_cores, _SC_INFO.num_subcores
_NW = _NC * _NS
_BPW = N_ROWS // _NW
_V_SC = V - V_TC
_RPW = _V_SC // _NW
_NBANDS = _RPW // 16

_sc_mesh = plsc.VectorSubcoreMesh(core_axis_name="c", subcore_axis_name="s")


@functools.partial(
    pl.kernel,
    mesh=_sc_mesh,
    out_type=[
        jax.ShapeDtypeStruct((N_ROWS, 16), jnp.float32),
        jax.ShapeDtypeStruct((_NW, N_ROWS), jnp.float32),
    ],
    scratch_types=[
        pltpu.VMEM((_BPW,), jnp.int32),
        pltpu.VMEM((_BPW, N_ROWS), jnp.float32),
        pltpu.VMEM((_BPW, 16), jnp.float32),
        pltpu.VMEM((16, N_ROWS), jnp.float32),
        pltpu.VMEM((16, N_ROWS), jnp.float32),
        pltpu.VMEM((N_ROWS,), jnp.float32),
        pltpu.SemaphoreType.DMA,
        pltpu.SemaphoreType.DMA,
        pltpu.SemaphoreType.DMA,
    ],
)
def _sc_pass(xt_hbm, tok_hbm, g_hbm, cs_hbm,
             tok_v, rows_v, g_v, buf0, buf1, acc_v, gsem, sem0, sem1):
    wid = lax.axis_index("s") * _NC + lax.axis_index("c")
    base = wid * _BPW
    lane = lax.iota(jnp.int32, 16)

    pltpu.sync_copy(tok_hbm.at[pl.ds(base, _BPW)], tok_v)
    gather = pltpu.async_copy(xt_hbm.at[tok_v], rows_v, gsem)

    row0 = V_TC + wid * _RPW
    for cc in range(N_ROWS // 16):
        acc_v[pl.ds(16 * cc, 16)] = jnp.zeros((16,), jnp.float32)

    def chunk_copy(b, buf, sem):
        return pltpu.make_async_copy(
            xt_hbm.at[pl.ds(row0 + 16 * b, 16), :], buf, sem)

    chunk_copy(0, buf0, sem0).start()
    chunk_copy(1, buf1, sem1).start()

    def accumulate(buf):
        for sub in range(2):
            for cc in range(N_ROWS // 16):
                s = buf[8 * sub, pl.ds(16 * cc, 16)]
                for r in range(1, 8):
                    s += buf[8 * sub + r, pl.ds(16 * cc, 16)]
                acc_v[pl.ds(16 * cc, 16)] += s

    def loop_body(m, _):
        b = 2 * m
        pltpu.make_async_copy(
            xt_hbm.at[pl.ds(row0, 16), :], buf0, sem0).wait()
        accumulate(buf0)

        @pl.when(b + 2 < _NBANDS)
        def _():
            chunk_copy(b + 2, buf0, sem0).start()

        pltpu.make_async_copy(
            xt_hbm.at[pl.ds(row0, 16), :], buf1, sem1).wait()
        accumulate(buf1)

        @pl.when(b + 3 < _NBANDS)
        def _():
            chunk_copy(b + 3, buf1, sem1).start()

        return 0

    lax.fori_loop(0, _NBANDS // 2, loop_body, 0)
    pltpu.sync_copy(acc_v, cs_hbm.at[wid])

    gather.wait()
    for k in range(_BPW):
        i_col = base + k
        vec = rows_v[k, pl.ds((i_col // 16) * 16, 16)]
        g_v.at[k][...] = jnp.where(lane == i_col % 16, vec, 0.0)
    pltpu.sync_copy(g_v, g_hbm.at[pl.ds(base, _BPW)])


def kernel(predicted_log_probabilities, tgt_tokens):
    n, v = predicted_log_probabilities.shape
    xt = predicted_log_probabilities.T
    tok_row = tgt_tokens.reshape(1, n).astype(jnp.float32)
    g16, cs = _sc_pass(xt, tgt_tokens)
    a = _tc_pass(tok_row, xt)
    notpad = (tgt_tokens != 0).astype(jnp.float32)
    g = jnp.sum(g16, axis=1)
    colsum = jnp.sum(cs, axis=0)
    return (a
            - EPS * jnp.sum(notpad * colsum)
            - (CONF - EPS) * jnp.sum(notpad * g))


# speedup vs baseline: 4.0512x; 1.0454x over previous
import functools
import math

import jax
import jax.numpy as jnp
from jax import lax
from jax.experimental import pallas as pl
from jax.experimental.pallas import tpu as pltpu, tpu_sc as plsc

V = 100000
N_ROWS = 1024
SMOOTH = 0.1
CONF = 1.0 - SMOOTH
EPS = SMOOTH / (V - 2)
K_ROW = CONF * math.log(CONF) + (V - 2) * EPS * math.log(EPS)

V_TC = 83616
VB = 2144


def _tc_body(tok_ref, xt_ref, out_ref):
    j = pl.program_id(0)
    c = tok_ref[...]
    notpad = (c != 0.0).astype(jnp.float32)
    xt = xt_ref[...]
    contrib = jnp.sum((-EPS * notpad) * xt)
    extra = jnp.sum(notpad * (K_ROW + EPS * xt[0:1, :]))
    contrib = contrib + jnp.where(j == 0, extra, 0.0)

    @pl.when(j == 0)
    def _init():
        out_ref[...] = jnp.zeros((1, 1), jnp.float32)

    out_ref[...] += jnp.full((1, 1), contrib, jnp.float32)


def _tc_pass(tok_row, xt):
    out = pl.pallas_call(
        _tc_body,
        grid=(V_TC // VB,),
        in_specs=[
            pl.BlockSpec((1, N_ROWS), lambda j: (0, 0)),
            pl.BlockSpec((VB, N_ROWS), lambda j: (j, 0)),
        ],
        out_specs=pl.BlockSpec((1, 1), lambda j: (0, 0)),
        out_shape=jax.ShapeDtypeStruct((1, 1), jnp.float32),
    )(tok_row, xt)
    return out[0, 0]


_SC_INFO = plsc.get_sparse_core_info()
_NC, _NS = _SC_INFO.num_cores, _SC_INFO.num_subcores
_NW = _NC * _NS
_BPW = N_ROWS // _NW
_V_SC = V - V_TC
_RPW = _V_SC // _NW
_NBANDS = _RPW // 16

_sc_mesh = plsc.VectorSubcoreMesh(core_axis_name="c", subcore_axis_name="s")


@functools.partial(
    pl.kernel,
    mesh=_sc_mesh,
    out_type=[
        jax.ShapeDtypeStruct((N_ROWS, 16), jnp.float32),
        jax.ShapeDtypeStruct((_NW, N_ROWS), jnp.float32),
    ],
    scratch_types=[
        pltpu.VMEM((_BPW,), jnp.int32),
        pltpu.VMEM((_BPW, N_ROWS), jnp.float32),
        pltpu.VMEM((_BPW, 16), jnp.float32),
        pltpu.VMEM((16, N_ROWS), jnp.float32),
        pltpu.VMEM((16, N_ROWS), jnp.float32),
        pltpu.VMEM((N_ROWS,), jnp.float32),
        pltpu.SemaphoreType.DMA,
        pltpu.SemaphoreType.DMA,
        pltpu.SemaphoreType.DMA,
    ],
)
def _sc_pass(xt_hbm, tok_hbm, g_hbm, cs_hbm,
             tok_v, rows_v, g_v, buf0, buf1, acc_v, gsem, sem0, sem1):
    wid = lax.axis_index("s") * _NC + lax.axis_index("c")
    base = wid * _BPW
    lane = lax.iota(jnp.int32, 16)

    pltpu.sync_copy(tok_hbm.at[pl.ds(base, _BPW)], tok_v)
    gather = pltpu.async_copy(xt_hbm.at[tok_v], rows_v, gsem)

    row0 = V_TC + wid * _RPW
    for cc in range(N_ROWS // 16):
        acc_v[pl.ds(16 * cc, 16)] = jnp.zeros((16,), jnp.float32)

    def chunk_copy(b, buf, sem):
        return pltpu.make_async_copy(
            xt_hbm.at[pl.ds(row0 + 16 * b, 16), :], buf, sem)

    chunk_copy(0, buf0, sem0).start()
    chunk_copy(1, buf1, sem1).start()

    def accumulate(buf):
        for sub in range(2):
            for cc in range(N_ROWS // 16):
                s = buf[8 * sub, pl.ds(16 * cc, 16)]
                for r in range(1, 8):
                    s += buf[8 * sub + r, pl.ds(16 * cc, 16)]
                acc_v[pl.ds(16 * cc, 16)] += s

    def loop_body(m, _):
        b = 2 * m
        pltpu.make_async_copy(
            xt_hbm.at[pl.ds(row0, 16), :], buf0, sem0).wait()
        accumulate(buf0)

        @pl.when(b + 2 < _NBANDS)
        def _():
            chunk_copy(b + 2, buf0, sem0).start()

        pltpu.make_async_copy(
            xt_hbm.at[pl.ds(row0, 16), :], buf1, sem1).wait()
        accumulate(buf1)

        @pl.when(b + 3 < _NBANDS)
        def _():
            chunk_copy(b + 3, buf1, sem1).start()

        return 0

    lax.fori_loop(0, _NBANDS // 2, loop_body, 0)
    pltpu.sync_copy(acc_v, cs_hbm.at[wid])

    gather.wait()
    for k in range(_BPW):
        i_col = base + k
        vec = rows_v[k, pl.ds((i_col // 16) * 16, 16)]
        g_v.at[k][...] = jnp.where(lane == i_col % 16, vec, 0.0)
    pltpu.sync_copy(g_v, g_hbm.at[pl.ds(base, _BPW)])


def kernel(predicted_log_probabilities, tgt_tokens):
    n, v = predicted_log_probabilities.shape
    xt = predicted_log_probabilities.T
    tok_row = tgt_tokens.reshape(1, n).astype(jnp.float32)
    g16, cs = _sc_pass(xt, tgt_tokens)
    a = _tc_pass(tok_row, xt)
    notpad = (tgt_tokens != 0).astype(jnp.float32)
    g = jnp.sum(g16, axis=1)
    colsum = jnp.sum(cs, axis=0)
    return (a
            - EPS * jnp.sum(notpad * colsum)
            - (CONF - EPS) * jnp.sum(notpad * g))


# speedup vs baseline: 4.2318x; 1.0446x over previous
import functools
import math

import jax
import jax.numpy as jnp
from jax import lax
from jax.experimental import pallas as pl
from jax.experimental.pallas import tpu as pltpu, tpu_sc as plsc

V = 100000
N_ROWS = 1024
SMOOTH = 0.1
CONF = 1.0 - SMOOTH
EPS = SMOOTH / (V - 2)
K_ROW = CONF * math.log(CONF) + (V - 2) * EPS * math.log(EPS)

V_TC = 84640
VB = 4232


def _tc_body(tok_ref, xt_ref, out_ref):
    j = pl.program_id(0)
    c = tok_ref[...]
    notpad = (c != 0.0).astype(jnp.float32)
    xt = xt_ref[...]
    contrib = jnp.sum((-EPS * notpad) * xt)
    extra = jnp.sum(notpad * (K_ROW + EPS * xt[0:1, :]))
    contrib = contrib + jnp.where(j == 0, extra, 0.0)

    @pl.when(j == 0)
    def _init():
        out_ref[...] = jnp.zeros((1, 1), jnp.float32)

    out_ref[...] += jnp.full((1, 1), contrib, jnp.float32)


def _tc_pass(tok_row, xt):
    out = pl.pallas_call(
        _tc_body,
        grid=(V_TC // VB,),
        in_specs=[
            pl.BlockSpec((1, N_ROWS), lambda j: (0, 0)),
            pl.BlockSpec((VB, N_ROWS), lambda j: (j, 0)),
        ],
        out_specs=pl.BlockSpec((1, 1), lambda j: (0, 0)),
        out_shape=jax.ShapeDtypeStruct((1, 1), jnp.float32),
    )(tok_row, xt)
    return out[0, 0]


_SC_INFO = plsc.get_sparse_core_info()
_NC, _NS = _SC_INFO.num_cores, _SC_INFO.num_subcores
_NW = _NC * _NS
_BPW = N_ROWS // _NW
_V_SC = V - V_TC
_RPW = _V_SC // _NW
_NBANDS = _RPW // 16

_sc_mesh = plsc.VectorSubcoreMesh(core_axis_name="c", subcore_axis_name="s")


@functools.partial(
    pl.kernel,
    mesh=_sc_mesh,
    out_type=[
        jax.ShapeDtypeStruct((N_ROWS, 16), jnp.float32),
        jax.ShapeDtypeStruct((_NW, N_ROWS), jnp.float32),
    ],
    scratch_types=[
        pltpu.VMEM((_BPW,), jnp.int32),
        pltpu.VMEM((_BPW, N_ROWS), jnp.float32),
        pltpu.VMEM((_BPW, 16), jnp.float32),
        pltpu.VMEM((16, N_ROWS), jnp.float32),
        pltpu.VMEM((16, N_ROWS), jnp.float32),
        pltpu.VMEM((N_ROWS,), jnp.float32),
        pltpu.SemaphoreType.DMA,
        pltpu.SemaphoreType.DMA,
        pltpu.SemaphoreType.DMA,
    ],
)
def _sc_pass(xt_hbm, tok_hbm, g_hbm, cs_hbm,
             tok_v, rows_v, g_v, buf0, buf1, acc_v, gsem, sem0, sem1):
    wid = lax.axis_index("s") * _NC + lax.axis_index("c")
    base = wid * _BPW
    lane = lax.iota(jnp.int32, 16)

    pltpu.sync_copy(tok_hbm.at[pl.ds(base, _BPW)], tok_v)
    gather = pltpu.async_copy(xt_hbm.at[tok_v], rows_v, gsem)

    row0 = V_TC + wid * _RPW
    for cc in range(N_ROWS // 16):
        acc_v[pl.ds(16 * cc, 16)] = jnp.zeros((16,), jnp.float32)

    def chunk_copy(b, buf, sem):
        return pltpu.make_async_copy(
            xt_hbm.at[pl.ds(row0 + 16 * b, 16), :], buf, sem)

    chunk_copy(0, buf0, sem0).start()
    chunk_copy(1, buf1, sem1).start()

    def accumulate(buf):
        for sub in range(2):
            for cc in range(N_ROWS // 16):
                s = buf[8 * sub, pl.ds(16 * cc, 16)]
                for r in range(1, 8):
                    s += buf[8 * sub + r, pl.ds(16 * cc, 16)]
                acc_v[pl.ds(16 * cc, 16)] += s

    def loop_body(m, _):
        b = 2 * m
        pltpu.make_async_copy(
            xt_hbm.at[pl.ds(row0, 16), :], buf0, sem0).wait()
        accumulate(buf0)

        @pl.when(b + 2 < _NBANDS)
        def _():
            chunk_copy(b + 2, buf0, sem0).start()

        pltpu.make_async_copy(
            xt_hbm.at[pl.ds(row0, 16), :], buf1, sem1).wait()
        accumulate(buf1)

        @pl.when(b + 3 < _NBANDS)
        def _():
            chunk_copy(b + 3, buf1, sem1).start()

        return 0

    lax.fori_loop(0, _NBANDS // 2, loop_body, 0)
    pltpu.sync_copy(acc_v, cs_hbm.at[wid])

    gather.wait()
    for k in range(_BPW):
        i_col = base + k
        vec = rows_v[k, pl.ds((i_col // 16) * 16, 16)]
        g_v.at[k][...] = jnp.where(lane == i_col % 16, vec, 0.0)
    pltpu.sync_copy(g_v, g_hbm.at[pl.ds(base, _BPW)])


def kernel(predicted_log_probabilities, tgt_tokens):
    n, v = predicted_log_probabilities.shape
    xt = predicted_log_probabilities.T
    tok_row = tgt_tokens.reshape(1, n).astype(jnp.float32)
    g16, cs = _sc_pass(xt, tgt_tokens)
    a = _tc_pass(tok_row, xt)
    notpad = (tgt_tokens != 0).astype(jnp.float32)
    g = jnp.sum(g16, axis=1)
    colsum = jnp.sum(cs, axis=0)
    return (a
            - EPS * jnp.sum(notpad * colsum)
            - (CONF - EPS) * jnp.sum(notpad * g))
